# baseline (speedup 1.0000x reference)
import functools

import jax
import jax.numpy as jnp
from jax import lax
from jax.experimental import pallas as pl
from jax.experimental.pallas import tpu as pltpu
from jax.experimental.pallas import tpu_sc as plsc

_NC = 2
_NS = 16
_W = _NC * _NS
_L = 16
_C = 80


def kernel(feat, edge_index):
    n_nodes, d = feat.shape
    e = edge_index.shape[1]
    per_w = e // _W
    n_chunks = per_w // _C
    assert per_w * _W == e and n_chunks * _C == per_w and d % (2 * _L) == 0
    assert n_chunks % 2 == 1

    packed = feat.astype(jnp.bfloat16)

    mesh = plsc.VectorSubcoreMesh(
        core_axis_name="c", subcore_axis_name="s",
        num_cores=_NC, num_subcores=_NS)

    @functools.partial(
        pl.kernel,
        out_type=jax.ShapeDtypeStruct((e,), jnp.float32),
        mesh=mesh,
        compiler_params=pltpu.CompilerParams(needs_layout_passes=False,
                                             use_tc_tiling_on_sc=False),
        scratch_types=[
            pltpu.VMEM((per_w,), jnp.int32),
            pltpu.VMEM((per_w,), jnp.int32),
            pltpu.VMEM((_C, d), jnp.bfloat16),
            pltpu.VMEM((_C, d), jnp.bfloat16),
            pltpu.VMEM((_C, d), jnp.bfloat16),
            pltpu.VMEM((_C, d), jnp.bfloat16),
            pltpu.VMEM((_C,), jnp.float32),
            pltpu.VMEM((_C,), jnp.float32),
            pltpu.VMEM_SHARED((n_nodes, d), jnp.bfloat16),
            pltpu.SemaphoreType.DMA,
            pltpu.SemaphoreType.DMA,
            pltpu.SemaphoreType.DMA,
            pltpu.SemaphoreType.DMA,
            pltpu.SemaphoreType.DMA,
            pltpu.SemaphoreType.DMA,
        ],
    )
    def ip_kernel(feat_h, edges_h, out_h, sidx, didx,
                  u0, v0, u1, v1, sc0, sc1, shared,
                  su0, sv0, su1, sv1, ss0, ss1):
        cid = lax.axis_index("c")
        sid = lax.axis_index("s")
        wid = sid * _NC + cid
        estart = wid * per_w

        @pl.when(sid == 0)
        def _():
            pltpu.sync_copy(feat_h, shared)

        pltpu.sync_copy(edges_h.at[0, pl.ds(estart, per_w)], sidx)
        pltpu.sync_copy(edges_h.at[1, pl.ds(estart, per_w)], didx)
        plsc.subcore_barrier()

        lanes = lax.iota(jnp.int32, _L)
        bufs = ((u0, v0, su0, sv0, sc0, ss0), (u1, v1, su1, sv1, sc1, ss1))

        def issue(c, b):
            ub, vb, su, sv, _, _ = bufs[b]
            pltpu.async_copy(feat_h.at[sidx.at[pl.ds(c * _C, _C)]], ub, su)
            pltpu.async_copy(shared.at[didx.at[pl.ds(c * _C, _C)]], vb, sv)

        def unpack2(x):
            return plsc.unpack(x, format=plsc.PackFormat.INTERLEAVED)

        def compute(c, b):
            ub, vb, su, sv, score, ss = bufs[b]
            pltpu.make_async_copy(
                feat_h.at[sidx.at[pl.ds(c * _C, _C)]], ub, su).wait()
            pltpu.make_async_copy(
                shared.at[didx.at[pl.ds(c * _C, _C)]], vb, sv).wait()
            pltpu.make_async_copy(
                out_h.at[pl.ds(estart, _C)], score, ss).wait()
            for g in range(_C // _L):
                tot = jnp.zeros((_L,), jnp.float32)
                for ee in range(_L):
                    row = g * _L + ee
                    acc = jnp.zeros((_L,), jnp.float32)
                    for k in range(d // (2 * _L)):
                        ua, ubb = unpack2(ub[row, pl.ds(k * 2 * _L, 2 * _L)])
                        va, vbb = unpack2(vb[row, pl.ds(k * 2 * _L, 2 * _L)])
                        acc = acc + ua * va + ubb * vbb
                    s = jnp.sum(acc)
                    tot = jnp.where(lanes == ee, s, tot)
                score[pl.ds(g * _L, _L)] = tot
            pltpu.async_copy(score, out_h.at[pl.ds(estart + c * _C, _C)], ss)

        pltpu.async_copy(sc0, out_h.at[pl.ds(estart + 0 * _C, _C)], ss0)
        pltpu.async_copy(sc1, out_h.at[pl.ds(estart + 1 * _C, _C)], ss1)
        issue(0, 0)

        def body(i, carry):
            c0 = 2 * i
            issue(c0 + 1, 1)
            compute(c0, 0)
            issue(c0 + 2, 0)
            compute(c0 + 1, 1)
            return c0 + 2

        last_c = lax.fori_loop(0, (n_chunks - 1) // 2, body, 0)
        compute(last_c, 0)

        pltpu.make_async_copy(out_h.at[pl.ds(estart, _C)], sc0, ss0).wait()
        pltpu.make_async_copy(out_h.at[pl.ds(estart, _C)], sc1, ss1).wait()

    return ip_kernel(packed, edge_index).reshape(e, 1)


# speedup vs baseline: 1.0147x; 1.0147x over previous
import functools

import jax
import jax.numpy as jnp
from jax import lax
from jax.experimental import pallas as pl
from jax.experimental.pallas import tpu as pltpu
from jax.experimental.pallas import tpu_sc as plsc

_NC = 2
_NS = 16
_W = _NC * _NS
_L = 16
_C = 80


def kernel(feat, edge_index):
    n_nodes, d = feat.shape
    e = edge_index.shape[1]
    per_w = e // _W
    n_chunks = per_w // _C
    assert per_w * _W == e and n_chunks * _C == per_w and d % (2 * _L) == 0
    assert n_chunks % 2 == 1

    packed = feat.astype(jnp.bfloat16)

    mesh = plsc.VectorSubcoreMesh(
        core_axis_name="c", subcore_axis_name="s",
        num_cores=_NC, num_subcores=_NS)

    @functools.partial(
        pl.kernel,
        out_type=jax.ShapeDtypeStruct((e,), jnp.float32),
        mesh=mesh,
        compiler_params=pltpu.CompilerParams(needs_layout_passes=False,
                                             use_tc_tiling_on_sc=False),
        scratch_types=[
            pltpu.VMEM((per_w,), jnp.int32),
            pltpu.VMEM((per_w,), jnp.int32),
            pltpu.VMEM((_C, d), jnp.bfloat16),
            pltpu.VMEM((_C, d), jnp.bfloat16),
            pltpu.VMEM((_C, d), jnp.bfloat16),
            pltpu.VMEM((_C, d), jnp.bfloat16),
            pltpu.VMEM((_C,), jnp.float32),
            pltpu.VMEM_SHARED((n_nodes, d), jnp.bfloat16),
            pltpu.SemaphoreType.DMA,
            pltpu.SemaphoreType.DMA,
            pltpu.SemaphoreType.DMA,
            pltpu.SemaphoreType.DMA,
        ],
    )
    def ip_kernel(feat_h, edges_h, out_h, sidx, didx,
                  u0, v0, u1, v1, score, shared, su0, sv0, su1, sv1):
        cid = lax.axis_index("c")
        sid = lax.axis_index("s")
        wid = sid * _NC + cid
        estart = wid * per_w

        @pl.when(sid == 0)
        def _():
            pltpu.sync_copy(feat_h, shared)

        pltpu.sync_copy(edges_h.at[0, pl.ds(estart, per_w)], sidx)
        pltpu.sync_copy(edges_h.at[1, pl.ds(estart, per_w)], didx)
        plsc.subcore_barrier()

        lanes = lax.iota(jnp.int32, _L)
        bufs = ((u0, v0, su0, sv0), (u1, v1, su1, sv1))

        def issue(c, b):
            ub, vb, su, sv = bufs[b]
            pltpu.async_copy(feat_h.at[sidx.at[pl.ds(c * _C, _C)]], ub, su)
            pltpu.async_copy(shared.at[didx.at[pl.ds(c * _C, _C)]], vb, sv)

        def unpack2(x):
            return plsc.unpack(x, format=plsc.PackFormat.INTERLEAVED)

        def compute(c, b):
            ub, vb, su, sv = bufs[b]
            pltpu.make_async_copy(
                feat_h.at[sidx.at[pl.ds(c * _C, _C)]], ub, su).wait()
            pltpu.make_async_copy(
                shared.at[didx.at[pl.ds(c * _C, _C)]], vb, sv).wait()
            for g in range(_C // _L):
                tot = jnp.zeros((_L,), jnp.float32)
                for ee in range(_L):
                    row = g * _L + ee
                    acc = jnp.zeros((_L,), jnp.float32)
                    for k in range(d // (2 * _L)):
                        ua, ubb = unpack2(ub[row, pl.ds(k * 2 * _L, 2 * _L)])
                        va, vbb = unpack2(vb[row, pl.ds(k * 2 * _L, 2 * _L)])
                        acc = acc + ua * va + ubb * vbb
                    s = jnp.sum(acc)
                    tot = jnp.where(lanes == ee, s, tot)
                score[pl.ds(g * _L, _L)] = tot
            pltpu.sync_copy(score, out_h.at[pl.ds(estart + c * _C, _C)])

        issue(0, 0)

        def body(i, carry):
            c0 = 2 * i
            issue(c0 + 1, 1)
            compute(c0, 0)
            issue(c0 + 2, 0)
            compute(c0 + 1, 1)
            return c0 + 2

        last_c = lax.fori_loop(0, (n_chunks - 1) // 2, body, 0)
        compute(last_c, 0)

    return ip_kernel(packed, edge_index).reshape(e, 1)
